# Initial kernel scaffold; baseline (speedup 1.0000x reference)
import jax
import jax.numpy as jnp
from jax.experimental import pallas as pl


def kernel(query, key, value, rel, timestamp, l1, l2, mask, Wq, bq, Wk, bk, Wv, bv):
    raise NotImplementedError("write your pallas kernel here")


# speedup vs baseline: 1.9385x; 1.9385x over previous
import functools

import jax
import jax.numpy as jnp
from jax.experimental import pallas as pl
from jax.experimental.pallas import tpu as pltpu

H = 16


def _proj_body(xq_ref, xk_ref, xv_ref, wq_ref, wk_ref, wv_ref,
               bq_ref, bk_ref, bv_ref, q_ref, k_ref, v_ref):
    dn = (((1,), (1,)), ((), ()))
    q_ref[...] = jax.lax.dot_general(
        xq_ref[...], wq_ref[...], dn,
        preferred_element_type=jnp.float32) + bq_ref[...]
    k_ref[...] = jax.lax.dot_general(
        xk_ref[...], wk_ref[...], dn,
        preferred_element_type=jnp.float32) + bk_ref[...]
    v_ref[...] = jax.lax.dot_general(
        xv_ref[...], wv_ref[...], dn,
        preferred_element_type=jnp.float32) + bv_ref[...]


def _attn_body(l1_ref, l2_ref, q_ref, k_ref, v_ref, rel_ref, ts_ref,
               out_ref, prob_ref, *, qb, s, hd):
    qi = pl.program_id(1)
    l1 = l1_ref[0, 0]
    l2 = l2_ref[0, 0]

    rows = jax.lax.broadcasted_iota(jnp.int32, (qb, s), 0) + qi * qb
    cols = jax.lax.broadcasted_iota(jnp.int32, (qb, s), 1)
    fut = cols > rows

    rel = rel_ref[0]
    relm = jnp.where(fut, rel, 0.0)
    rl = jnp.where(relm == 0.0, jnp.float32(-10000.0), relm)
    rmax = jnp.max(rl, axis=-1, keepdims=True)
    re = jnp.exp(rl - rmax)
    rel_attn = re / jnp.sum(re, axis=-1, keepdims=True)

    ts = ts_ref[0]
    tv = jnp.exp(-jnp.abs(ts))
    tl = jnp.where(fut, -jnp.inf, tv)
    tmax = jnp.max(tl, axis=-1, keepdims=True)
    te = jnp.exp(tl - tmax)
    time_attn = te / jnp.sum(te, axis=-1, keepdims=True)

    shared = ((1.0 - l1) * l2) * time_attn + l1 * rel_attn
    p_scale = (1.0 - l1) * (1.0 - l2)
    scale = jnp.float32(1.0 / (hd ** 0.5))

    for h in range(H):
        qh = q_ref[0, :, h * hd:(h + 1) * hd]
        kh = k_ref[0, :, h * hd:(h + 1) * hd]
        sc = jax.lax.dot_general(
            qh, kh, (((1,), (1,)), ((), ())),
            preferred_element_type=jnp.float32) * scale
        sc = jnp.where(fut, jnp.float32(-1e9), sc)
        smax = jnp.max(sc, axis=-1, keepdims=True)
        se = jnp.exp(sc - smax)
        p = se / jnp.sum(se, axis=-1, keepdims=True)
        p = p_scale * p + shared
        prob_ref[0, h] = p
        vh = v_ref[0, :, h * hd:(h + 1) * hd]
        out_ref[0, :, h * hd:(h + 1) * hd] = jnp.dot(
            p, vh, preferred_element_type=jnp.float32)


def kernel(query, key, value, rel, timestamp, l1, l2, mask,
           Wq, bq, Wk, bk, Wv, bv):
    b, s, d = query.shape
    hd = d // H
    bs = b * s
    mb = 512
    qb = 128

    xq = query.reshape(bs, d)
    xk = key.reshape(bs, d)
    xv = value.reshape(bs, d)
    bq2 = bq.reshape(1, d)
    bk2 = bk.reshape(1, d)
    bv2 = bv.reshape(1, d)

    row_spec = pl.BlockSpec((mb, d), lambda i: (i, 0))
    w_spec = pl.BlockSpec((d, d), lambda i: (0, 0))
    b_spec = pl.BlockSpec((1, d), lambda i: (0, 0))
    q2, k2, v2 = pl.pallas_call(
        _proj_body,
        grid=(bs // mb,),
        in_specs=[row_spec, row_spec, row_spec, w_spec, w_spec, w_spec,
                  b_spec, b_spec, b_spec],
        out_specs=[row_spec, row_spec, row_spec],
        out_shape=[jax.ShapeDtypeStruct((bs, d), jnp.float32)] * 3,
        compiler_params=pltpu.CompilerParams(
            dimension_semantics=("parallel",),
            vmem_limit_bytes=56 * 1024 * 1024,
        ),
    )(xq, xk, xv, Wq, Wk, Wv, bq2, bk2, bv2)

    q3 = q2.reshape(b, s, d)
    k3 = k2.reshape(b, s, d)
    v3 = v2.reshape(b, s, d)
    l1s = l1.reshape(1, 1)
    l2s = l2.reshape(1, 1)

    body = functools.partial(_attn_body, qb=qb, s=s, hd=hd)
    smem_spec = pl.BlockSpec(memory_space=pltpu.SMEM)
    out, prob = pl.pallas_call(
        body,
        grid=(b, s // qb),
        in_specs=[
            smem_spec, smem_spec,
            pl.BlockSpec((1, qb, d), lambda bi, qi: (bi, qi, 0)),
            pl.BlockSpec((1, s, d), lambda bi, qi: (bi, 0, 0)),
            pl.BlockSpec((1, s, d), lambda bi, qi: (bi, 0, 0)),
            pl.BlockSpec((1, qb, s), lambda bi, qi: (bi, qi, 0)),
            pl.BlockSpec((1, qb, s), lambda bi, qi: (bi, qi, 0)),
        ],
        out_specs=[
            pl.BlockSpec((1, qb, d), lambda bi, qi: (bi, qi, 0)),
            pl.BlockSpec((1, H, qb, s), lambda bi, qi: (bi, 0, qi, 0)),
        ],
        out_shape=[
            jax.ShapeDtypeStruct((b, s, d), jnp.float32),
            jax.ShapeDtypeStruct((b, H, s, s), jnp.float32),
        ],
        compiler_params=pltpu.CompilerParams(
            dimension_semantics=("parallel", "arbitrary"),
            vmem_limit_bytes=56 * 1024 * 1024,
        ),
    )(l1s, l2s, q3, k3, v3, rel, timestamp)

    return out, prob
